# baseline (speedup 1.0000x reference)
import math

import jax
import jax.numpy as jnp
import numpy as np
from jax import lax
from jax.experimental import pallas as pl
from jax.experimental.pallas import tpu as pltpu
from jax.experimental.pallas import tpu_sc as plsc

NUM_PLAYERS = 5
NUM_MOVES = 50
EMB_DIM = NUM_PLAYERS * NUM_MOVES
GAMMA = 0.9
LN_GAMMA = math.log(GAMMA)
ACTIVE_T = 1024
NUM_WORKERS = 16
CHUNK = ACTIVE_T // NUM_WORKERS
LANES = 16
PAD_DIM = 256

_ZERO_ACC = np.zeros((LANES * PAD_DIM,), dtype=np.float32)


def _sc_body(mv_hbm, pv_hbm, off_hbm, zero_hbm, out_hbm,
             mv_v, pv_v, off_v, acc, red, tmp, outv, shared, sem):
    sid = lax.axis_index("s")
    lane = lax.iota(jnp.int32, LANES)

    base = sid * CHUNK
    cpz = pltpu.async_copy(zero_hbm, acc, sem)
    cp1 = pltpu.async_copy(mv_hbm.at[pl.ds(base, CHUNK)], mv_v, sem)
    cp2 = pltpu.async_copy(pv_hbm.at[pl.ds(base, CHUNK)], pv_v, sem)
    cp3 = pltpu.async_copy(off_hbm, off_v, sem)
    cpz.wait()
    cp1.wait()
    cp2.wait()
    cp3.wait()

    off = off_v[...]
    row_base = lane * PAD_DIM
    for j in range(CHUNK // LANES):
        mv = mv_v[pl.ds(j * LANES, LANES)]
        pv = pv_v[pl.ds(j * LANES, LANES)]
        adj = jnp.where(pv >= off, pv - off, pv + (NUM_PLAYERS - off))
        ids = adj * NUM_MOVES + mv
        t = (base + j * LANES + lane).astype(jnp.float32)
        fac = jnp.exp(t * jnp.float32(LN_GAMMA))
        plsc.addupdate_scatter(acc, [row_base + ids], fac)

    for c in range(PAD_DIM // LANES):
        s = acc[pl.ds(c * LANES, LANES)]
        for r in range(1, LANES):
            s = s + acc[pl.ds(r * PAD_DIM + c * LANES, LANES)]
        red[pl.ds(c * LANES, LANES)] = s
    pltpu.sync_copy(red, shared.at[pl.ds(sid * PAD_DIM, PAD_DIM)])

    plsc.subcore_barrier()

    pltpu.sync_copy(shared, tmp)
    col = sid * LANES
    s = tmp[pl.ds(col, LANES)]
    for r in range(1, NUM_WORKERS):
        s = s + tmp[pl.ds(r * PAD_DIM + col, LANES)]
    outv[...] = s

    @pl.when(sid < NUM_WORKERS - 1)
    def _full_chunk():
        pltpu.sync_copy(outv, out_hbm.at[pl.ds(col, LANES)])

    @pl.when(sid == NUM_WORKERS - 1)
    def _tail_chunk():
        tail = EMB_DIM - (NUM_WORKERS - 1) * LANES
        pltpu.sync_copy(outv.at[pl.ds(0, tail)],
                        out_hbm.at[pl.ds((NUM_WORKERS - 1) * LANES, tail)])


def kernel(move_ids, players, cur_player_offset):
    mv = move_ids.astype(jnp.int32)
    pv = players.astype(jnp.int32)
    off = jnp.full((LANES,), cur_player_offset, dtype=jnp.int32)
    mesh = plsc.VectorSubcoreMesh(
        core_axis_name="c", subcore_axis_name="s", num_cores=1
    )
    f = pl.kernel(
        _sc_body,
        out_type=jax.ShapeDtypeStruct((EMB_DIM,), jnp.float32),
        mesh=mesh,
        compiler_params=pltpu.CompilerParams(needs_layout_passes=False),
        scratch_types=[
            pltpu.VMEM((CHUNK,), jnp.int32),
            pltpu.VMEM((CHUNK,), jnp.int32),
            pltpu.VMEM((LANES,), jnp.int32),
            pltpu.VMEM((LANES * PAD_DIM,), jnp.float32),
            pltpu.VMEM((PAD_DIM,), jnp.float32),
            pltpu.VMEM((NUM_WORKERS * PAD_DIM,), jnp.float32),
            pltpu.VMEM((LANES,), jnp.float32),
            pltpu.VMEM_SHARED((NUM_WORKERS * PAD_DIM,), jnp.float32),
            pltpu.SemaphoreType.DMA,
        ],
    )
    return f(mv, pv, off, _ZERO_ACC)


# speedup vs baseline: 1.0456x; 1.0456x over previous
import math

import jax
import jax.numpy as jnp
from jax import lax
from jax.experimental import pallas as pl
from jax.experimental.pallas import tpu as pltpu
from jax.experimental.pallas import tpu_sc as plsc

NUM_PLAYERS = 5
NUM_MOVES = 50
EMB_DIM = NUM_PLAYERS * NUM_MOVES
GAMMA = 0.9
LN_GAMMA = math.log(GAMMA)
ACTIVE_T = 1024
NUM_WORKERS = 16
CHUNK = ACTIVE_T // NUM_WORKERS
LANES = 16
PAD_DIM = 256


def _sc_body(mv_hbm, pv_hbm, off_hbm, zero_hbm, out_hbm,
             mv_v, pv_v, off_v, acc, red, tmp, outv, shared, sem):
    sid = lax.axis_index("s")
    lane = lax.iota(jnp.int32, LANES)

    base = sid * CHUNK
    cpz = pltpu.async_copy(zero_hbm, acc, sem)
    cp1 = pltpu.async_copy(mv_hbm.at[pl.ds(base, CHUNK)], mv_v, sem)
    cp2 = pltpu.async_copy(pv_hbm.at[pl.ds(base, CHUNK)], pv_v, sem)
    cp3 = pltpu.async_copy(off_hbm, off_v, sem)
    cpz.wait()
    cp1.wait()
    cp2.wait()
    cp3.wait()

    off = off_v[...]
    row_base = lane * PAD_DIM
    for j in range(CHUNK // LANES):
        mv = mv_v[pl.ds(j * LANES, LANES)]
        pv = pv_v[pl.ds(j * LANES, LANES)]
        adj = jnp.where(pv >= off, pv - off, pv + (NUM_PLAYERS - off))
        ids = adj * NUM_MOVES + mv
        t = (base + j * LANES + lane).astype(jnp.float32)
        fac = jnp.exp(t * jnp.float32(LN_GAMMA))
        plsc.addupdate_scatter(acc, [row_base + ids], fac)

    n_chunks = PAD_DIM // LANES

    def _row_add(r, carry):
        rb = r * PAD_DIM
        return tuple(
            carry[c] + acc[pl.ds(rb + c * LANES, LANES)] for c in range(n_chunks)
        )

    init = tuple(acc[pl.ds(c * LANES, LANES)] for c in range(n_chunks))
    sums = lax.fori_loop(1, LANES, _row_add, init)
    for c in range(n_chunks):
        red[pl.ds(c * LANES, LANES)] = sums[c]
    pltpu.sync_copy(red, shared.at[pl.ds(sid * PAD_DIM, PAD_DIM)])

    plsc.subcore_barrier()

    pltpu.sync_copy(shared, tmp)
    col = sid * LANES

    def _part_add(r, s):
        return s + tmp[pl.ds(r * PAD_DIM + col, LANES)]

    outv[...] = lax.fori_loop(1, NUM_WORKERS, _part_add, tmp[pl.ds(col, LANES)])

    @pl.when(sid < NUM_WORKERS - 1)
    def _full_chunk():
        pltpu.sync_copy(outv, out_hbm.at[pl.ds(col, LANES)])

    @pl.when(sid == NUM_WORKERS - 1)
    def _tail_chunk():
        tail = EMB_DIM - (NUM_WORKERS - 1) * LANES
        pltpu.sync_copy(outv.at[pl.ds(0, tail)],
                        out_hbm.at[pl.ds((NUM_WORKERS - 1) * LANES, tail)])


def kernel(move_ids, players, cur_player_offset):
    mv = move_ids.astype(jnp.int32)
    pv = players.astype(jnp.int32)
    off = jnp.full((LANES,), cur_player_offset, dtype=jnp.int32)
    zero = jnp.zeros((LANES * PAD_DIM,), dtype=jnp.float32)
    mesh = plsc.VectorSubcoreMesh(
        core_axis_name="c", subcore_axis_name="s", num_cores=1
    )
    f = pl.kernel(
        _sc_body,
        out_type=jax.ShapeDtypeStruct((EMB_DIM,), jnp.float32),
        mesh=mesh,
        compiler_params=pltpu.CompilerParams(needs_layout_passes=False),
        scratch_types=[
            pltpu.VMEM((CHUNK,), jnp.int32),
            pltpu.VMEM((CHUNK,), jnp.int32),
            pltpu.VMEM((LANES,), jnp.int32),
            pltpu.VMEM((LANES * PAD_DIM,), jnp.float32),
            pltpu.VMEM((PAD_DIM,), jnp.float32),
            pltpu.VMEM((NUM_WORKERS * PAD_DIM,), jnp.float32),
            pltpu.VMEM((LANES,), jnp.float32),
            pltpu.VMEM_SHARED((NUM_WORKERS * PAD_DIM,), jnp.float32),
            pltpu.SemaphoreType.DMA,
        ],
    )
    return f(mv, pv, off, zero)
